# Initial kernel scaffold; baseline (speedup 1.0000x reference)
import jax
import jax.numpy as jnp
from jax.experimental import pallas as pl


def kernel(token_ids, weights):
    raise NotImplementedError("write your pallas kernel here")


# speedup vs baseline: 1.7878x; 1.7878x over previous
import functools

import jax
import jax.numpy as jnp
from jax import lax
from jax.experimental import pallas as pl
from jax.experimental.pallas import tpu as pltpu
from jax.experimental.pallas import tpu_sc as plsc

NUM_CORES = 2
NUM_SUBCORES = 16
NUM_WORKERS = NUM_CORES * NUM_SUBCORES
CHUNK = 512


@functools.partial(jax.jit, static_argnums=(2, 3))
def _gather_rows(flat_ids, weights, B, D):
    per_w = B // NUM_WORKERS
    nsteps = per_w // CHUNK
    mesh = plsc.VectorSubcoreMesh(core_axis_name="c", subcore_axis_name="s")

    @functools.partial(
        pl.kernel,
        out_type=jax.ShapeDtypeStruct((B, D), jnp.float32),
        mesh=mesh,
        scratch_types=[
            pltpu.VMEM((CHUNK,), jnp.int32),
            pltpu.VMEM((CHUNK, D), jnp.float32),
            pltpu.SemaphoreType.DMA,
        ],
        compiler_params=pltpu.CompilerParams(use_tc_tiling_on_sc=False),
    )
    def gather_kernel(idx_hbm, table_hbm, out_hbm, idx_v, rows_v, sem):
        wid = lax.axis_index("s") * NUM_CORES + lax.axis_index("c")
        base = wid * per_w

        def step(g, carry):
            off = base + g * CHUNK
            pltpu.sync_copy(idx_hbm.at[pl.ds(off, CHUNK)], idx_v)
            pltpu.async_copy(table_hbm.at[idx_v], rows_v, sem).wait()
            pltpu.sync_copy(rows_v, out_hbm.at[pl.ds(off, CHUNK)])
            return carry

        lax.fori_loop(0, nsteps, step, 0)

    return gather_kernel(flat_ids, weights)


def kernel(token_ids, weights):
    B = token_ids.shape[0] * token_ids.shape[1]
    D = weights.shape[1]
    flat = token_ids.reshape(B).astype(jnp.int32)
    out = _gather_rows(flat, weights, B, D)
    return out.reshape(*token_ids.shape, D)


# speedup vs baseline: 1.8664x; 1.0440x over previous
import functools

import jax
import jax.numpy as jnp
from jax import lax
from jax.experimental import pallas as pl
from jax.experimental.pallas import tpu as pltpu
from jax.experimental.pallas import tpu_sc as plsc

NUM_CORES = 2
NUM_SUBCORES = 16
NUM_WORKERS = NUM_CORES * NUM_SUBCORES
CHUNK = 512
NBUF = 2


@functools.partial(jax.jit, static_argnums=(2, 3))
def _gather_rows(flat_ids, weights, B, D):
    per_w = B // NUM_WORKERS
    nsteps = per_w // CHUNK
    assert nsteps % NBUF == 0
    mesh = plsc.VectorSubcoreMesh(core_axis_name="c", subcore_axis_name="s")

    @functools.partial(
        pl.kernel,
        out_type=jax.ShapeDtypeStruct((B, D), jnp.float32),
        mesh=mesh,
        scratch_types=[
            pltpu.VMEM((per_w,), jnp.int32),
            pltpu.VMEM((NBUF, CHUNK, D), jnp.float32),
            pltpu.SemaphoreType.DMA,
            pltpu.SemaphoreType.DMA,
            pltpu.SemaphoreType.DMA,
            pltpu.SemaphoreType.DMA,
        ],
        compiler_params=pltpu.CompilerParams(use_tc_tiling_on_sc=False),
    )
    def gather_kernel(idx_hbm, table_hbm, out_hbm, idx_v, rows_v,
                      gs0, gs1, ws0, ws1):
        gsem = (gs0, gs1)
        wsem = (ws0, ws1)
        wid = lax.axis_index("s") * NUM_CORES + lax.axis_index("c")
        base = wid * per_w
        pltpu.sync_copy(idx_hbm.at[pl.ds(base, per_w)], idx_v)

        def start_gather(g, b):
            pltpu.async_copy(
                table_hbm.at[idx_v.at[pl.ds(g * CHUNK, CHUNK)]],
                rows_v.at[b], gsem[b])

        def start_write(g, b):
            pltpu.async_copy(
                rows_v.at[b], out_hbm.at[pl.ds(base + g * CHUNK, CHUNK)],
                wsem[b])

        def wait_write(b):
            pltpu.make_async_copy(
                rows_v.at[b], out_hbm.at[pl.ds(base, CHUNK)], wsem[b]).wait()

        def wait_gather(b):
            pltpu.make_async_copy(
                table_hbm.at[idx_v.at[pl.ds(0, CHUNK)]],
                rows_v.at[b], gsem[b]).wait()

        start_gather(0, 0)

        def group(i, carry):
            for b in range(NBUF):
                g = i * NBUF + b
                nb = (b + 1) % NBUF

                @pl.when((g + 1 < nsteps) & (g >= 1))
                def _():
                    wait_write(nb)

                @pl.when(g + 1 < nsteps)
                def _():
                    start_gather(g + 1, nb)

                wait_gather(b)
                start_write(g, b)
            return carry

        lax.fori_loop(0, nsteps // NBUF, group, 0)
        for b in range(NBUF):
            wait_write(b)

    return gather_kernel(flat_ids, weights)


def kernel(token_ids, weights):
    B = token_ids.shape[0] * token_ids.shape[1]
    D = weights.shape[1]
    flat = token_ids.reshape(B).astype(jnp.int32)
    out = _gather_rows(flat, weights, B, D)
    return out.reshape(*token_ids.shape, D)
